# Initial kernel scaffold; baseline (speedup 1.0000x reference)
import jax
import jax.numpy as jnp
from jax.experimental import pallas as pl


def kernel(x, edge_index, batch, W, b):
    raise NotImplementedError("write your pallas kernel here")


# speedup vs baseline: 1.7132x; 1.7132x over previous
import functools

import jax
import jax.numpy as jnp
from jax import lax
from jax.experimental import pallas as pl
from jax.experimental.pallas import tpu as pltpu
from jax.experimental.pallas import tpu_sc as plsc

_LANES = 1024
_SC_CORES = 2
_SC_SUBCORES = 16


def _scores_body(x_ref, w_ref, b_ref, o_ref, *, n_rows, block_rows):
    i = pl.program_id(0)
    xb = x_ref[...]
    wt = w_ref[...]
    s = jnp.dot(xb, wt, preferred_element_type=jnp.float32)
    s = jnp.tanh(s + b_ref[0, 0])
    gidx = i * block_rows + jax.lax.broadcasted_iota(
        jnp.int32, (block_rows, 1), 0)
    o_ref[...] = jnp.where(gidx < n_rows, s, -jnp.inf)


def _compute_scores(x, W, b):
    n, d = x.shape
    block_rows = 8192
    grid = (n + block_rows - 1) // block_rows
    out = pl.pallas_call(
        functools.partial(_scores_body, n_rows=n, block_rows=block_rows),
        grid=(grid,),
        in_specs=[
            pl.BlockSpec((block_rows, d), lambda i: (i, 0)),
            pl.BlockSpec((d, 1), lambda i: (0, 0)),
            pl.BlockSpec((1, 1), lambda i: (0, 0)),
        ],
        out_specs=pl.BlockSpec((block_rows, 1), lambda i: (i, 0)),
        out_shape=jax.ShapeDtypeStruct((grid * block_rows, 1), jnp.float32),
    )(x, W.T.reshape(d, 1), b.reshape(1, 1))
    return out.reshape(grid * block_rows)


def _bitonic_pairs_body(k_ref, i_ref, ok_ref, oi_ref, *, rows, lanes):
    n = rows * lanes
    K = k_ref[...]
    I = i_ref[...]
    sub = jax.lax.broadcasted_iota(jnp.int32, (rows, lanes), 0)
    lane = jax.lax.broadcasted_iota(jnp.int32, (rows, lanes), 1)
    half = lanes // 2
    kk = 2
    while kk <= n:
        if kk <= half:
            asc = (lane & kk) == 0
        else:
            asc = (sub & (kk // lanes)) == 0
        j = kk // 2
        while j >= 1:
            if j <= half:
                hi = (lane & j) != 0
                pk = jnp.where(hi, jnp.roll(K, j, 1), jnp.roll(K, -j, 1))
                pi = jnp.where(hi, jnp.roll(I, j, 1), jnp.roll(I, -j, 1))
            else:
                m = j // lanes
                hi = (sub & m) != 0
                pk = jnp.where(hi, jnp.roll(K, m, 0), jnp.roll(K, -m, 0))
                pi = jnp.where(hi, jnp.roll(I, m, 0), jnp.roll(I, -m, 0))
            pre = (K > pk) | ((K == pk) & (I < pi))
            keep = pre == (asc ^ hi)
            K = jnp.where(keep, K, pk)
            I = jnp.where(keep, I, pi)
            j //= 2
        kk *= 2
    ok_ref[...] = K
    oi_ref[...] = I


def _sort_pairs(keys_lin, idx_lin, rows, lanes):
    K = keys_lin.reshape(rows, lanes)
    I = idx_lin.reshape(rows, lanes)
    ok, oi = pl.pallas_call(
        functools.partial(_bitonic_pairs_body, rows=rows, lanes=lanes),
        out_shape=(
            jax.ShapeDtypeStruct((rows, lanes), jnp.float32),
            jax.ShapeDtypeStruct((rows, lanes), jnp.int32),
        ),
    )(K, I)
    return ok.reshape(rows * lanes), oi.reshape(rows * lanes)


def _bitonic_int_body(k_ref, ok_ref, *, rows, lanes):
    n = rows * lanes
    K = k_ref[...]
    sub = jax.lax.broadcasted_iota(jnp.int32, (rows, lanes), 0)
    lane = jax.lax.broadcasted_iota(jnp.int32, (rows, lanes), 1)
    half = lanes // 2
    kk = 2
    while kk <= n:
        if kk <= half:
            asc = (lane & kk) == 0
        else:
            asc = (sub & (kk // lanes)) == 0
        j = kk // 2
        while j >= 1:
            if j <= half:
                hi = (lane & j) != 0
                pk = jnp.where(hi, jnp.roll(K, j, 1), jnp.roll(K, -j, 1))
            else:
                m = j // lanes
                hi = (sub & m) != 0
                pk = jnp.where(hi, jnp.roll(K, m, 0), jnp.roll(K, -m, 0))
            pre = K < pk
            keep = pre == (asc ^ hi)
            K = jnp.where(keep, K, pk)
            j //= 2
        kk *= 2
    ok_ref[...] = K


def _sort_ints(keys_lin, rows, lanes):
    K = keys_lin.reshape(rows, lanes)
    ok = pl.pallas_call(
        functools.partial(_bitonic_int_body, rows=rows, lanes=lanes),
        out_shape=jax.ShapeDtypeStruct((rows, lanes), jnp.int32),
    )(K)
    return ok.reshape(rows * lanes)


def _sc_gather_rows(x, idx, chunk=112):
    n, d = x.shape
    b = idx.shape[0]
    nw = _SC_CORES * _SC_SUBCORES
    quantum = nw * chunk
    b_pad = ((b + quantum - 1) // quantum) * quantum
    if b_pad != b:
        idx = jnp.concatenate(
            [idx, jnp.zeros((b_pad - b,), jnp.int32)])
    b_per_w = b_pad // nw
    n_ch = b_per_w // chunk
    mesh = plsc.VectorSubcoreMesh(
        core_axis_name="c", subcore_axis_name="s",
        num_cores=_SC_CORES, num_subcores=_SC_SUBCORES)

    def body(x_hbm, idx_hbm, out_hbm, iv0, iv1, rv0, rv1, si0, si1, sd0, sd1):
        wid = lax.axis_index("s") * _SC_CORES + lax.axis_index("c")
        iv, rv, si, sd = (iv0, iv1), (rv0, rv1), (si0, si1), (sd0, sd1)

        def sl(i):
            return pl.ds(wid * b_per_w + i * chunk, chunk)

        hidx = [None] * n_ch
        hst = [None] * n_ch
        for i in range(min(2, n_ch)):
            hidx[i] = pltpu.async_copy(idx_hbm.at[sl(i)], iv[i % 2], si[i % 2])
        for i in range(n_ch):
            p = i % 2
            hidx[i].wait()
            if i >= 2:
                hst[i - 2].wait()
            g = pltpu.async_copy(x_hbm.at[iv[p]], rv[p], sd[p])
            g.wait()
            if i + 2 < n_ch:
                hidx[i + 2] = pltpu.async_copy(
                    idx_hbm.at[sl(i + 2)], iv[p], si[p])
            hst[i] = pltpu.async_copy(rv[p], out_hbm.at[sl(i)], sd[p])
        for h in hst[max(0, n_ch - 2):]:
            h.wait()

    out = pl.kernel(
        body,
        out_type=jax.ShapeDtypeStruct((b_pad, d), jnp.float32),
        mesh=mesh,
        scratch_types=[
            pltpu.VMEM((chunk,), jnp.int32),
            pltpu.VMEM((chunk,), jnp.int32),
            pltpu.VMEM((chunk, d), jnp.float32),
            pltpu.VMEM((chunk, d), jnp.float32),
            pltpu.SemaphoreType.DMA,
            pltpu.SemaphoreType.DMA,
            pltpu.SemaphoreType.DMA,
            pltpu.SemaphoreType.DMA,
        ],
    )(x, idx)
    return out[:b]


def _sc_mesh():
    return plsc.VectorSubcoreMesh(
        core_axis_name="c", subcore_axis_name="s",
        num_cores=_SC_CORES, num_subcores=_SC_SUBCORES)


def _pad_to(a, size, fill):
    if a.shape[0] == size:
        return a
    return jnp.concatenate(
        [a, jnp.full((size - a.shape[0],), fill, a.dtype)])


def _sc_build_new_mask(perm, keep_idx, ranks, n, chunk=112):
    nw = _SC_CORES * _SC_SUBCORES
    nc = perm.shape[0]
    nk = keep_idx.shape[0]
    quantum = nw * chunk
    b_pad = ((max(nc, nk) + quantum - 1) // quantum) * quantum
    dead = b_pad - nc + b_pad - nk
    n_pad = ((n + dead + 7) // 8) * 8
    pad_a = jnp.arange(n, n + b_pad - nc, dtype=jnp.int32)
    pad_b = jnp.arange(n + b_pad - nc, n + dead, dtype=jnp.int32)
    idx_all = jnp.concatenate([perm, pad_a, keep_idx, pad_b])
    val_all = jnp.concatenate(
        [ranks, jnp.zeros((b_pad - nc, ), jnp.int32),
         jnp.zeros((b_pad, ), jnp.int32)])
    b_per_w = (2 * b_pad) // nw
    n_ch = b_per_w // chunk

    def body(idx_hbm, val_hbm, out_hbm,
             iv0, iv1, vv0, vv1, si0, si1, sv0, sv1, ss0, ss1):
        wid = lax.axis_index("s") * _SC_CORES + lax.axis_index("c")
        iv, vv = (iv0, iv1), (vv0, vv1)
        si, sv, ss = (si0, si1), (sv0, sv1), (ss0, ss1)

        def sl(i):
            return pl.ds(wid * b_per_w + i * chunk, chunk)

        hs = [None] * n_ch
        for i in range(n_ch):
            p = i % 2
            if i >= 2:
                hs[i - 2].wait()
            hi = pltpu.async_copy(idx_hbm.at[sl(i)], iv[p], si[p])
            hv = pltpu.async_copy(val_hbm.at[sl(i)], vv[p], sv[p])
            hi.wait()
            hv.wait()
            hs[i] = pltpu.async_copy(vv[p], out_hbm.at[iv[p]], ss[p])
        for i in range(max(0, n_ch - 2), n_ch):
            hs[i].wait()

    out = pl.kernel(
        body,
        out_type=jax.ShapeDtypeStruct((n_pad,), jnp.int32),
        mesh=_sc_mesh(),
        scratch_types=[
            pltpu.VMEM((chunk,), jnp.int32),
            pltpu.VMEM((chunk,), jnp.int32),
            pltpu.VMEM((chunk,), jnp.int32),
            pltpu.VMEM((chunk,), jnp.int32),
            pltpu.SemaphoreType.DMA,
            pltpu.SemaphoreType.DMA,
            pltpu.SemaphoreType.DMA,
            pltpu.SemaphoreType.DMA,
            pltpu.SemaphoreType.DMA,
            pltpu.SemaphoreType.DMA,
        ],
    )(idx_all, val_all)
    return out


def _sc_edge_batch(new_mask, e_i, e_j, keep_idx, batch, perm, chunk=112):
    nw = _SC_CORES * _SC_SUBCORES
    bk = keep_idx.shape[0]
    quantum = nw * chunk
    b_pad = ((bk + quantum - 1) // quantum) * quantum
    keep_p = _pad_to(keep_idx, b_pad, 0)
    perm_p = _pad_to(perm, b_pad, 0)
    b_per_w = b_pad // nw
    n_ch = b_per_w // chunk

    def body(nm_hbm, ei_hbm, ej_hbm, keep_hbm, batch_hbm, perm_hbm,
             oi_hbm, oj_hbm, ob_hbm,
             k_v, p_v, a_v, c_v, oi_v, oj_v, ob_v,
             s_k, s_p, s_i, s_j, s_b, s_si, s_sj, s_sb):
        wid = lax.axis_index("s") * _SC_CORES + lax.axis_index("c")

        def sl(i):
            return pl.ds(wid * b_per_w + i * chunk, chunk)

        hk = pltpu.async_copy(keep_hbm.at[sl(0)], k_v, s_k)
        hp = pltpu.async_copy(perm_hbm.at[sl(0)], p_v, s_p)
        st_i = st_j = st_b = None
        for i in range(n_ch):
            hk.wait()
            g_ei = pltpu.async_copy(ei_hbm.at[k_v], a_v, s_i)
            g_ej = pltpu.async_copy(ej_hbm.at[k_v], c_v, s_j)
            hp.wait()
            if st_b is not None:
                st_b.wait()
            g_b = pltpu.async_copy(batch_hbm.at[p_v], ob_v, s_b)
            g_ei.wait()
            if st_i is not None:
                st_i.wait()
            g_ni = pltpu.async_copy(nm_hbm.at[a_v], oi_v, s_i)
            g_ej.wait()
            if st_j is not None:
                st_j.wait()
            g_nj = pltpu.async_copy(nm_hbm.at[c_v], oj_v, s_j)
            if i + 1 < n_ch:
                hk = pltpu.async_copy(keep_hbm.at[sl(i + 1)], k_v, s_k)
            g_b.wait()
            st_b = pltpu.async_copy(ob_v, ob_hbm.at[sl(i)], s_sb)
            if i + 1 < n_ch:
                hp = pltpu.async_copy(perm_hbm.at[sl(i + 1)], p_v, s_p)
            g_ni.wait()
            st_i = pltpu.async_copy(oi_v, oi_hbm.at[sl(i)], s_si)
            g_nj.wait()
            st_j = pltpu.async_copy(oj_v, oj_hbm.at[sl(i)], s_sj)
        st_b.wait()
        st_i.wait()
        st_j.wait()

    oi, oj, ob = pl.kernel(
        body,
        out_type=(
            jax.ShapeDtypeStruct((b_pad,), jnp.int32),
            jax.ShapeDtypeStruct((b_pad,), jnp.int32),
            jax.ShapeDtypeStruct((b_pad,), jnp.int32),
        ),
        mesh=_sc_mesh(),
        scratch_types=(
            [pltpu.VMEM((chunk,), jnp.int32)] * 7
            + [pltpu.SemaphoreType.DMA] * 8
        ),
    )(new_mask, e_i, e_j, keep_p, batch, perm_p)
    return oi[:bk], oj[:bk], ob[:bk]


def _next_pow2(n):
    p = 1
    while p < n:
        p *= 2
    return p


def kernel(x, edge_index, batch, W, b):
    n, d = x.shape
    nc = max(int(n * 0.5), 1)

    scores = _compute_scores(x, W, b)

    n_sort = _next_pow2(n)
    if n_sort < scores.shape[0]:
        n_sort = _next_pow2(scores.shape[0])
    pad = n_sort - scores.shape[0]
    if pad:
        scores = jnp.concatenate(
            [scores, jnp.full((pad,), -jnp.inf, jnp.float32)])
    idx = jnp.arange(n_sort, dtype=jnp.int32)
    lanes = min(_LANES, n_sort)
    rows = n_sort // lanes
    svals, sidx = _sort_pairs(scores, idx, rows, lanes)

    perm = sidx[:nc]
    scores_pool = svals[:nc]

    rej = sidx[nc:n]
    n_keep = n - nc
    n2 = _next_pow2(n_keep)
    pad2 = n2 - n_keep
    if pad2:
        rej = jnp.concatenate(
            [rej, jnp.full((pad2,), jnp.iinfo(jnp.int32).max, jnp.int32)])
    lanes2 = min(_LANES, n2)
    rows2 = n2 // lanes2
    keep_idx = _sort_ints(rej, rows2, lanes2)[:n_keep]

    x_pool = _sc_gather_rows(x, perm)
    ranks = jnp.arange(nc, dtype=jnp.int32)
    new_mask = _sc_build_new_mask(perm, keep_idx, ranks, n, chunk=128)
    oi, oj, ob = _sc_edge_batch(
        new_mask, edge_index[0], edge_index[1], keep_idx, batch, perm)
    edge_index_pool = jnp.stack([oi, oj], axis=0)

    return (x_pool, edge_index_pool, perm, ob, scores_pool)
